# baseline (speedup 1.0000x reference)
import functools

import jax
import jax.numpy as jnp
from jax import lax
from jax.experimental import pallas as pl
from jax.experimental.pallas import tpu as pltpu
from jax.experimental.pallas import tpu_sc as plsc

VOCAB = 100000
DIM = 128
B = 4096
S = 200

NC = 2
NS = 16
NW = NC * NS

HALF = S // 2
SEQ_PER_W = B // NW
NBUF = 3
NPHASE = 2
PSEQ = SEQ_PER_W // NPHASE


def _sc_body(ids_hbm, table_hbm, pos_hbm, out_hbm, idx_v, pos_v, bufs, gsems, osems):
    wid = lax.axis_index("s") * NC + lax.axis_index("c")
    seq0 = wid * SEQ_PER_W

    pltpu.sync_copy(pos_hbm, pos_v)

    def add_pos(b):
        buf = bufs[b]

        def add_body(r, _):
            for j in range(DIM // 16):
                sl = pl.ds(j * 16, 16)
                plsc.addupdate(buf.at[r, sl], pos_v[r, sl])
            return ()

        lax.fori_loop(0, S, add_body, (), unroll=4)

    for phase in range(NPHASE):
        pbase = phase * PSEQ

        pltpu.sync_copy(
            ids_hbm.at[pl.ds((seq0 + pbase) * 2, 2 * PSEQ)], idx_v)

        def start_gather(q, b):
            pltpu.async_copy(table_hbm.at[idx_v.at[2 * q]],
                             bufs[b].at[pl.ds(0, HALF)], gsems[b])
            pltpu.async_copy(table_hbm.at[idx_v.at[2 * q + 1]],
                             bufs[b].at[pl.ds(HALF, HALF)], gsems[b])

        def wait_gather(q, b):
            pltpu.make_async_copy(table_hbm.at[idx_v.at[2 * q]],
                                  bufs[b].at[pl.ds(0, HALF)], gsems[b]).wait()
            pltpu.make_async_copy(table_hbm.at[idx_v.at[2 * q + 1]],
                                  bufs[b].at[pl.ds(HALF, HALF)], gsems[b]).wait()

        def start_out(q, b):
            pltpu.async_copy(bufs[b], out_hbm.at[seq0 + pbase + q], osems[b])

        def wait_out(q, b):
            pltpu.make_async_copy(
                bufs[b], out_hbm.at[seq0 + pbase + q], osems[b]).wait()

        start_gather(0, 0)
        start_gather(1, 1)

        def group_body(g, _):
            for b in range(NBUF):
                q = g * NBUF + b
                bn = (b + NBUF - 1) % NBUF

                @pl.when(q >= 1)
                def _wait_prev_out():
                    wait_out(q - 1, bn)

                @pl.when(q + 2 < PSEQ)
                def _fire():
                    start_gather(q + 2, bn)

                wait_gather(q, b)
                add_pos(b)
                start_out(q, b)
            return ()

        lax.fori_loop(0, (PSEQ - 1) // NBUF, group_body, ())

        wait_out(PSEQ - 2, 2)
        wait_gather(PSEQ - 1, 0)
        add_pos(0)
        start_out(PSEQ - 1, 0)
        wait_out(PSEQ - 1, 0)


@functools.partial(jax.jit, static_argnames=())
def kernel(token_ids, token_table, pos_table):
    ids = token_ids.astype(jnp.int32).reshape(2 * B, HALF)

    mesh = plsc.VectorSubcoreMesh(
        core_axis_name="c", subcore_axis_name="s", num_cores=NC,
        num_subcores=NS)
    return pl.kernel(
        _sc_body,
        out_type=jax.ShapeDtypeStruct((B, S, DIM), jnp.float32),
        mesh=mesh,
        scratch_types=[
            pltpu.VMEM((2 * PSEQ, HALF), jnp.int32),
            pltpu.VMEM((S, DIM), jnp.float32),
            [pltpu.VMEM((S, DIM), jnp.float32) for _ in range(NBUF)],
            [pltpu.SemaphoreType.DMA for _ in range(NBUF)],
            [pltpu.SemaphoreType.DMA for _ in range(NBUF)],
        ],
    )(ids, token_table, pos_table)


# speedup vs baseline: 1.2118x; 1.2118x over previous
import functools

import jax
import jax.numpy as jnp
from jax import lax
from jax.experimental import pallas as pl
from jax.experimental.pallas import tpu as pltpu
from jax.experimental.pallas import tpu_sc as plsc

VOCAB = 100000
DIM = 128
B = 4096
S = 200

NC = 2
NS = 16
NW = NC * NS

HALF = S // 2
SEQ_PER_W = B // NW
NBUF = 3
NPHASE = 2
PSEQ = SEQ_PER_W // NPHASE


def _sc_body(ids_hbm, table_hbm, pos_hbm, out_hbm, idx_v, pos_v, bufs, gsems, osems):
    wid = lax.axis_index("s") * NC + lax.axis_index("c")
    seq0 = wid * SEQ_PER_W

    pltpu.sync_copy(pos_hbm, pos_v)

    def add_pos(b):
        buf = bufs[b]

        def add_body(r, _):
            for j in range(DIM // 16):
                sl = pl.ds(j * 16, 16)
                plsc.addupdate(buf.at[r, sl], pos_v[r, sl])
            return ()

        lax.fori_loop(0, S, add_body, (), unroll=8)

    for phase in range(NPHASE):
        pbase = phase * PSEQ

        pltpu.sync_copy(
            ids_hbm.at[pl.ds((seq0 + pbase) * 2, 2 * PSEQ)], idx_v)

        def start_gather(q, b):
            pltpu.async_copy(table_hbm.at[idx_v.at[2 * q]],
                             bufs[b].at[pl.ds(0, HALF)], gsems[b])
            pltpu.async_copy(table_hbm.at[idx_v.at[2 * q + 1]],
                             bufs[b].at[pl.ds(HALF, HALF)], gsems[b])

        def wait_gather(q, b):
            pltpu.make_async_copy(table_hbm.at[idx_v.at[2 * q]],
                                  bufs[b].at[pl.ds(0, HALF)], gsems[b]).wait()
            pltpu.make_async_copy(table_hbm.at[idx_v.at[2 * q + 1]],
                                  bufs[b].at[pl.ds(HALF, HALF)], gsems[b]).wait()

        def start_out(q, b):
            pltpu.async_copy(bufs[b], out_hbm.at[seq0 + pbase + q], osems[b])

        def wait_out(q, b):
            pltpu.make_async_copy(
                bufs[b], out_hbm.at[seq0 + pbase + q], osems[b]).wait()

        start_gather(0, 0)
        start_gather(1, 1)

        def group_body(g, _):
            for b in range(NBUF):
                q = g * NBUF + b
                bn = (b + NBUF - 1) % NBUF

                wait_gather(q, b)
                add_pos(b)
                start_out(q, b)

                @pl.when(q >= 1)
                def _wait_prev_out():
                    wait_out(q - 1, bn)

                @pl.when(q + 2 < PSEQ)
                def _fire():
                    start_gather(q + 2, bn)
            return ()

        lax.fori_loop(0, (PSEQ - 1) // NBUF, group_body, ())

        wait_gather(PSEQ - 1, 0)
        add_pos(0)
        start_out(PSEQ - 1, 0)
        wait_out(PSEQ - 2, 2)
        wait_out(PSEQ - 1, 0)


@functools.partial(jax.jit, static_argnames=())
def kernel(token_ids, token_table, pos_table):
    ids = token_ids.astype(jnp.int32).reshape(2 * B, HALF)

    mesh = plsc.VectorSubcoreMesh(
        core_axis_name="c", subcore_axis_name="s", num_cores=NC,
        num_subcores=NS)
    return pl.kernel(
        _sc_body,
        out_type=jax.ShapeDtypeStruct((B, S, DIM), jnp.float32),
        mesh=mesh,
        scratch_types=[
            pltpu.VMEM((2 * PSEQ, HALF), jnp.int32),
            pltpu.VMEM((S, DIM), jnp.float32),
            [pltpu.VMEM((S, DIM), jnp.float32) for _ in range(NBUF)],
            [pltpu.SemaphoreType.DMA for _ in range(NBUF)],
            [pltpu.SemaphoreType.DMA for _ in range(NBUF)],
        ],
    )(ids, token_table, pos_table)
